# Initial kernel scaffold; baseline (speedup 1.0000x reference)
import jax
import jax.numpy as jnp
from jax.experimental import pallas as pl


def kernel(pos_u, pos_v, neg_v, u_embs_w, v_embs_w):
    raise NotImplementedError("write your pallas kernel here")


# speedup vs baseline: 1.6575x; 1.6575x over previous
import functools

import jax
import jax.numpy as jnp
from jax import lax
from jax.experimental import pallas as pl
from jax.experimental.pallas import tpu as pltpu
from jax.experimental.pallas import tpu_sc as plsc

_EMB_DIM = 64
_N_NEG = 5
_CH = 32


def _sc_scores(pos_u, pos_v, neg_flat, u_w, v_w):
    B = pos_u.shape[0]
    NC, NS, L = 2, 16, 16
    NW = NC * NS
    bpw = B // NW
    nch = bpw // _CH
    nk = _EMB_DIM // L
    mesh = plsc.VectorSubcoreMesh(core_axis_name="c", subcore_axis_name="s",
                                  num_cores=NC, num_subcores=NS)

    NT = 1 + _N_NEG

    @functools.partial(
        pl.kernel,
        out_type=jax.ShapeDtypeStruct((NT, B), jnp.float32),
        mesh=mesh,
        compiler_params=pltpu.CompilerParams(needs_layout_passes=False,
                                             use_tc_tiling_on_sc=False),
        scratch_types=[
            pltpu.VMEM((_CH,), jnp.int32),
            pltpu.VMEM((_CH,), jnp.int32),
            pltpu.VMEM((_CH * _N_NEG,), jnp.int32),
            pltpu.VMEM((_CH, _EMB_DIM), jnp.float32),
            pltpu.VMEM((_CH, _EMB_DIM), jnp.float32),
            pltpu.VMEM((_CH * _N_NEG, _EMB_DIM), jnp.float32),
            pltpu.VMEM((NT * L * L,), jnp.float32),
            pltpu.VMEM((NT, _CH), jnp.float32),
            pltpu.SemaphoreType.DMA,
            pltpu.SemaphoreType.DMA,
            pltpu.SemaphoreType.DMA,
        ],
    )
    def sc_kernel(pos_u_h, pos_v_h, neg_h, u_w_h, v_w_h, out_h,
                  u_idx, v_idx, n_idx, u_rows, v_rows, n_rows, scr, sbuf,
                  s0, s1, s2):
        wid = lax.axis_index("s") * NC + lax.axis_index("c")
        base = wid * bpw
        base16 = lax.iota(jnp.int32, L) * L

        def chunk_body(c, carry):
            off = base + c * _CH
            pltpu.sync_copy(pos_u_h.at[pl.ds(off, _CH)], u_idx)
            pltpu.sync_copy(pos_v_h.at[pl.ds(off, _CH)], v_idx)
            pltpu.sync_copy(neg_h.at[pl.ds(off * _N_NEG, _CH * _N_NEG)], n_idx)
            cu = pltpu.async_copy(u_w_h.at[u_idx], u_rows, s0)
            cv = pltpu.async_copy(v_w_h.at[v_idx], v_rows, s1)
            cn = pltpu.async_copy(v_w_h.at[n_idx], n_rows, s2)
            cu.wait()
            cv.wait()
            cn.wait()
            for g in range(_CH // L):
                for i in range(L):
                    e = g * L + i
                    u_vecs = [u_rows[e, pl.ds(k * L, L)] for k in range(nk)]
                    for t in range(NT):
                        if t == 0:
                            rows, r = v_rows, e
                        else:
                            rows, r = n_rows, e * _N_NEG + (t - 1)
                        acc = u_vecs[0] * rows[r, pl.ds(0, L)]
                        for k in range(1, nk):
                            acc = acc + u_vecs[k] * rows[r, pl.ds(k * L, L)]
                        scr[pl.ds((t * L + i) * L, L)] = acc
                for t in range(NT):
                    res = plsc.load_gather(scr, [base16 + (t * L * L)])
                    for j in range(1, L):
                        res = res + plsc.load_gather(scr, [base16 + (t * L * L + j)])
                    if t > 0:
                        res = -res
                    sbuf[t, pl.ds(g * L, L)] = res
            pltpu.sync_copy(sbuf, out_h.at[:, pl.ds(off, _CH)])
            return carry

        lax.fori_loop(0, nch, chunk_body, 0)

    return sc_kernel(pos_u, pos_v, neg_flat, u_w, v_w)


def _tc_loss(scores2d):
    def body(x_ref, o_ref):
        x = x_ref[...]
        ls = jnp.minimum(x, 0.0) - jnp.log1p(jnp.exp(-jnp.abs(x)))
        o_ref[0, 0] = -jnp.sum(ls)

    return pl.pallas_call(
        body,
        out_shape=jax.ShapeDtypeStruct((1, 1), jnp.float32),
        out_specs=pl.BlockSpec(memory_space=pltpu.SMEM),
    )(scores2d)


def kernel(pos_u, pos_v, neg_v, u_embs_w, v_embs_w):
    B = pos_u.shape[0]
    neg_flat = neg_v.astype(jnp.int32).reshape(-1)
    scores = _sc_scores(pos_u.astype(jnp.int32), pos_v.astype(jnp.int32),
                        neg_flat, u_embs_w, v_embs_w)
    scores2d = scores.reshape((1 + _N_NEG) * B // 128, 128)
    return _tc_loss(scores2d)[0, 0]


# speedup vs baseline: 2.3725x; 1.4314x over previous
import functools

import jax
import jax.numpy as jnp
from jax import lax
from jax.experimental import pallas as pl
from jax.experimental.pallas import tpu as pltpu
from jax.experimental.pallas import tpu_sc as plsc

_EMB_DIM = 64
_N_NEG = 5
_CH = 32


def _sc_scores(pos_u, pos_v, neg_flat, u_w, v_w):
    B = pos_u.shape[0]
    NC, NS, L = 2, 16, 16
    NW = NC * NS
    bpw = B // NW
    nch = bpw // _CH
    nk = _EMB_DIM // L
    NT = 1 + _N_NEG
    mesh = plsc.VectorSubcoreMesh(core_axis_name="c", subcore_axis_name="s",
                                  num_cores=NC, num_subcores=NS)

    @functools.partial(
        pl.kernel,
        out_type=jax.ShapeDtypeStruct((NT * B,), jnp.float32),
        mesh=mesh,
        compiler_params=pltpu.CompilerParams(needs_layout_passes=False),
        scratch_types=[
            pltpu.VMEM((_CH,), jnp.int32),
            pltpu.VMEM((_CH,), jnp.int32),
            pltpu.VMEM((_CH * _N_NEG,), jnp.int32),
            pltpu.VMEM((_CH, _EMB_DIM), jnp.float32),
            pltpu.VMEM((_CH, _EMB_DIM), jnp.float32),
            pltpu.VMEM((_CH * _N_NEG, _EMB_DIM), jnp.float32),
            pltpu.VMEM((NT * L * L,), jnp.float32),
            pltpu.VMEM((NT, _CH), jnp.float32),
            pltpu.SemaphoreType.DMA,
            pltpu.SemaphoreType.DMA,
            pltpu.SemaphoreType.DMA,
        ],
    )
    def sc_kernel(pos_u_h, pos_v_h, neg_h, u_w_h, v_w_h, out_h,
                  u_idx, v_idx, n_idx, u_rows, v_rows, n_rows, scr, sbuf,
                  s0, s1, s2):
        wid = lax.axis_index("s") * NC + lax.axis_index("c")
        base = wid * bpw
        base16 = lax.iota(jnp.int32, L) * L

        def chunk_body(c, carry):
            off = base + c * _CH
            pltpu.sync_copy(pos_u_h.at[pl.ds(off, _CH)], u_idx)
            pltpu.sync_copy(pos_v_h.at[pl.ds(off, _CH)], v_idx)
            pltpu.sync_copy(neg_h.at[pl.ds(off * _N_NEG, _CH * _N_NEG)], n_idx)
            copies = []
            for g in range(_CH // L):
                uvec = u_idx[pl.ds(g * L, L)]
                vvec = v_idx[pl.ds(g * L, L)]
                for j in range(L):
                    e = g * L + j
                    copies.append(pltpu.async_copy(
                        u_w_h.at[uvec[j]], u_rows.at[e], s0))
                    copies.append(pltpu.async_copy(
                        v_w_h.at[vvec[j]], v_rows.at[e], s1))
            for g in range(_CH * _N_NEG // L):
                nvec = n_idx[pl.ds(g * L, L)]
                for j in range(L):
                    e = g * L + j
                    copies.append(pltpu.async_copy(
                        v_w_h.at[nvec[j]], n_rows.at[e], s2))
            for cp in copies:
                cp.wait()
            for g in range(_CH // L):
                for i in range(L):
                    e = g * L + i
                    u_vecs = [u_rows[e, pl.ds(k * L, L)] for k in range(nk)]
                    for t in range(NT):
                        if t == 0:
                            rows, r = v_rows, e
                        else:
                            rows, r = n_rows, e * _N_NEG + (t - 1)
                        acc = u_vecs[0] * rows[r, pl.ds(0, L)]
                        for k in range(1, nk):
                            acc = acc + u_vecs[k] * rows[r, pl.ds(k * L, L)]
                        scr[pl.ds((t * L + i) * L, L)] = acc
                for t in range(NT):
                    res = plsc.load_gather(scr, [base16 + (t * L * L)])
                    for j in range(1, L):
                        res = res + plsc.load_gather(scr, [base16 + (t * L * L + j)])
                    if t > 0:
                        res = -res
                    sbuf[t, pl.ds(g * L, L)] = res
            for t in range(NT):
                pltpu.sync_copy(sbuf.at[t, pl.ds(0, _CH)],
                                out_h.at[pl.ds(t * B + off, _CH)])
            return carry

        lax.fori_loop(0, nch, chunk_body, 0)

    return sc_kernel(pos_u, pos_v, neg_flat, u_w, v_w)


def _tc_loss(scores2d):
    def body(x_ref, o_ref):
        x = x_ref[...]
        ls = jnp.minimum(x, 0.0) - jnp.log1p(jnp.exp(-jnp.abs(x)))
        o_ref[0, 0] = -jnp.sum(ls)

    return pl.pallas_call(
        body,
        out_shape=jax.ShapeDtypeStruct((1, 1), jnp.float32),
        out_specs=pl.BlockSpec(memory_space=pltpu.SMEM),
    )(scores2d)


def kernel(pos_u, pos_v, neg_v, u_embs_w, v_embs_w):
    B = pos_u.shape[0]
    neg_flat = neg_v.astype(jnp.int32).reshape(-1)
    scores = _sc_scores(pos_u.astype(jnp.int32), pos_v.astype(jnp.int32),
                        neg_flat, u_embs_w, v_embs_w)
    scores2d = scores.reshape((1 + _N_NEG) * B // 128, 128)
    return _tc_loss(scores2d)[0, 0]


# speedup vs baseline: 2.3755x; 1.0013x over previous
import functools

import jax
import jax.numpy as jnp
from jax import lax
from jax.experimental import pallas as pl
from jax.experimental.pallas import tpu as pltpu
from jax.experimental.pallas import tpu_sc as plsc

_EMB_DIM = 64
_N_NEG = 5
_CH = 32


def _sc_scores(pos_u, pos_v, neg_flat, u_w, v_w):
    B = pos_u.shape[0]
    NC, NS, L = 2, 16, 16
    NW = NC * NS
    bpw = B // NW
    nch = bpw // _CH
    nk = _EMB_DIM // L
    NT = 1 + _N_NEG
    mesh = plsc.VectorSubcoreMesh(core_axis_name="c", subcore_axis_name="s",
                                  num_cores=NC, num_subcores=NS)

    @functools.partial(
        pl.kernel,
        out_type=jax.ShapeDtypeStruct((NT * B,), jnp.float32),
        mesh=mesh,
        compiler_params=pltpu.CompilerParams(needs_layout_passes=False,
                                             use_tc_tiling_on_sc=True),
        scratch_types=[
            pltpu.VMEM((_CH,), jnp.int32),
            pltpu.VMEM((_CH,), jnp.int32),
            pltpu.VMEM((_CH * _N_NEG,), jnp.int32),
            pltpu.VMEM((_CH, _EMB_DIM), jnp.float32),
            pltpu.VMEM((_CH, _EMB_DIM), jnp.float32),
            pltpu.VMEM((_CH * _N_NEG, _EMB_DIM), jnp.float32),
            pltpu.VMEM((NT * L * L,), jnp.float32),
            pltpu.VMEM((NT, _CH), jnp.float32),
            pltpu.SemaphoreType.DMA,
            pltpu.SemaphoreType.DMA,
            pltpu.SemaphoreType.DMA,
        ],
    )
    def sc_kernel(pos_u_h, pos_v_h, neg_h, u_w_h, v_w_h, out_h,
                  u_idx, v_idx, n_idx, u_rows, v_rows, n_rows, scr, sbuf,
                  s0, s1, s2):
        wid = lax.axis_index("s") * NC + lax.axis_index("c")
        base = wid * bpw
        base16 = lax.iota(jnp.int32, L) * L

        def chunk_body(c, carry):
            off = base + c * _CH
            pltpu.sync_copy(pos_u_h.at[pl.ds(off, _CH)], u_idx)
            pltpu.sync_copy(pos_v_h.at[pl.ds(off, _CH)], v_idx)
            pltpu.sync_copy(neg_h.at[pl.ds(off * _N_NEG, _CH * _N_NEG)], n_idx)
            copies = []
            for g in range(_CH // L):
                uvec = u_idx[pl.ds(g * L, L)]
                vvec = v_idx[pl.ds(g * L, L)]
                for j in range(L):
                    e = g * L + j
                    copies.append(pltpu.async_copy(
                        u_w_h.at[uvec[j]], u_rows.at[e], s0))
                    copies.append(pltpu.async_copy(
                        v_w_h.at[vvec[j]], v_rows.at[e], s1))
            for g in range(_CH * _N_NEG // L):
                nvec = n_idx[pl.ds(g * L, L)]
                for j in range(L):
                    e = g * L + j
                    copies.append(pltpu.async_copy(
                        v_w_h.at[nvec[j]], n_rows.at[e], s2))
            for cp in copies:
                cp.wait()
            for g in range(_CH // L):
                for i in range(L):
                    e = g * L + i
                    u_vecs = [u_rows[e, pl.ds(k * L, L)] for k in range(nk)]
                    for t in range(NT):
                        if t == 0:
                            rows, r = v_rows, e
                        else:
                            rows, r = n_rows, e * _N_NEG + (t - 1)
                        acc = u_vecs[0] * rows[r, pl.ds(0, L)]
                        for k in range(1, nk):
                            acc = acc + u_vecs[k] * rows[r, pl.ds(k * L, L)]
                        scr[pl.ds((t * L + i) * L, L)] = acc
                for t in range(NT):
                    res = plsc.load_gather(scr, [base16 + (t * L * L)])
                    for j in range(1, L):
                        res = res + plsc.load_gather(scr, [base16 + (t * L * L + j)])
                    if t > 0:
                        res = -res
                    sbuf[t, pl.ds(g * L, L)] = res
            for t in range(NT):
                pltpu.sync_copy(sbuf.at[t, pl.ds(0, _CH)],
                                out_h.at[pl.ds(t * B + off, _CH)])
            return carry

        lax.fori_loop(0, nch, chunk_body, 0)

    return sc_kernel(pos_u, pos_v, neg_flat, u_w, v_w)


def _tc_loss(scores2d):
    def body(x_ref, o_ref):
        x = x_ref[...]
        ls = jnp.minimum(x, 0.0) - jnp.log1p(jnp.exp(-jnp.abs(x)))
        o_ref[0, 0] = -jnp.sum(ls)

    return pl.pallas_call(
        body,
        out_shape=jax.ShapeDtypeStruct((1, 1), jnp.float32),
        out_specs=pl.BlockSpec(memory_space=pltpu.SMEM),
    )(scores2d)


def kernel(pos_u, pos_v, neg_v, u_embs_w, v_embs_w):
    B = pos_u.shape[0]
    neg_flat = neg_v.astype(jnp.int32).reshape(-1)
    scores = _sc_scores(pos_u.astype(jnp.int32), pos_v.astype(jnp.int32),
                        neg_flat, u_embs_w, v_embs_w)
    scores2d = scores.reshape((1 + _N_NEG) * B // 128, 128)
    return _tc_loss(scores2d)[0, 0]
